# Initial kernel scaffold; baseline (speedup 1.0000x reference)
import jax
import jax.numpy as jnp
from jax.experimental import pallas as pl


def kernel(z, proj_w, proj_b, embed_w):
    raise NotImplementedError("write your pallas kernel here")


# speedup vs baseline: 1.2981x; 1.2981x over previous
import functools

import jax
import jax.numpy as jnp
from jax.experimental import pallas as pl
from jax.experimental.pallas import tpu as pltpu
from jax.experimental.pallas import tpu_sc as plsc

_KT = 1024


def _qa_body(nk, kt_last, z_ref, pw_ref, pb_ref, emb_ref,
             ft_out, f2_out, lse_out, ind_out, diff_out,
             ft_s, f2_s, m_s, s_s, bv_s, bi_s, acc_s):
    b = pl.program_id(0)
    kt = pl.program_id(1)
    nb = pl.num_programs(0)
    hw = ft_s.shape[1]

    @pl.when(kt == 0)
    def _init():
        ft = jnp.dot(pw_ref[...], z_ref[0],
                     preferred_element_type=jnp.float32) + pb_ref[...]
        ft_s[...] = ft
        ft_out[...] = ft
        f2 = jnp.sum(ft * ft, axis=0, keepdims=True)
        f2_s[...] = f2
        f2_out[0] = f2
        m_s[...] = jnp.full((1, hw), -jnp.inf, jnp.float32)
        s_s[...] = jnp.zeros((1, hw), jnp.float32)
        bv_s[...] = jnp.full((1, hw), -jnp.inf, jnp.float32)
        bi_s[...] = jnp.zeros((1, hw), jnp.int32)

    e = emb_ref[...]
    e2 = jnp.sum(e * e, axis=1, keepdims=True)
    mm = jnp.dot(e, ft_s[...], preferred_element_type=jnp.float32)
    v = -((f2_s[...] - 2.0 * mm) + e2)

    tmax = jnp.max(v, axis=0, keepdims=True)
    rows = jax.lax.broadcasted_iota(jnp.int32, v.shape, 0)
    big = jnp.int32(nk * _KT)
    idx = jnp.min(jnp.where(v == tmax, rows, big), axis=0, keepdims=True)
    idx = idx + kt * _KT

    m_old = m_s[...]
    m_new = jnp.maximum(m_old, tmax)
    s_s[...] = (s_s[...] * jnp.exp(m_old - m_new)
                + jnp.sum(jnp.exp(v - m_new), axis=0, keepdims=True))
    m_s[...] = m_new

    flip = tmax > bv_s[...]
    bi_s[...] = jnp.where(flip, idx, bi_s[...])
    bv_s[...] = jnp.maximum(bv_s[...], tmax)

    @pl.when(kt == kt_last)
    def _fin():
        lse_out[0] = m_s[...] + jnp.log(s_s[...])
        ind_out[0] = bi_s[...]
        part = -jnp.sum(bv_s[...]).reshape(1, 1)
        tot = jnp.where(b == 0, part, acc_s[...] + part)
        acc_s[...] = tot

        @pl.when(b == nb - 1)
        def _done():
            n_total = nb * hw * ft_s.shape[0]
            diff_out[...] = tot * jnp.float32(12.5 / n_total)


def _qb_body(emb_ref, ft_ref, f2_ref, lse_ref, lp_out):
    e = emb_ref[...]
    e2 = jnp.sum(e * e, axis=1, keepdims=True)
    mm = jnp.dot(e, ft_ref[...], preferred_element_type=jnp.float32)
    dist = (f2_ref[0] - 2.0 * mm) + e2
    lp_out[0] = (-dist) - lse_ref[0]


def _qd_body(zq_ref, zqt_out):
    zqt_out[0] = zq_ref[...].T


def _gather_rows(ind2, embed_w):
    nrow, lanes = ind2.shape
    n = nrow * lanes
    k, d = embed_w.shape
    nw = 32
    bpw = n // nw
    chunks = bpw // lanes
    mesh = plsc.VectorSubcoreMesh(core_axis_name="c", subcore_axis_name="s")

    @functools.partial(
        pl.kernel,
        out_type=jax.ShapeDtypeStruct((n, d), jnp.float32),
        mesh=mesh,
        scratch_types=[
            pltpu.VMEM((chunks, lanes), jnp.int32),
            pltpu.VMEM((bpw, d), jnp.float32),
            pltpu.SemaphoreType.DMA,
        ],
    )
    def gk(idx_hbm, tab_hbm, out_hbm, idx_v, rows_v, sem):
        wid = jax.lax.axis_index("s") * 2 + jax.lax.axis_index("c")
        pltpu.sync_copy(idx_hbm.at[pl.ds(wid * chunks, chunks)], idx_v)
        cps = [
            pltpu.async_copy(tab_hbm.at[idx_v.at[j]],
                             rows_v.at[pl.ds(j * lanes, lanes)], sem)
            for j in range(chunks)
        ]
        for cp in cps:
            cp.wait()
        pltpu.sync_copy(rows_v, out_hbm.at[pl.ds(wid * bpw, bpw)])

    return gk(ind2, embed_w)


def kernel(z, proj_w, proj_b, embed_w):
    bz, c, h, w = z.shape
    d = proj_w.shape[0]
    k = embed_w.shape[0]
    hw = h * w
    n = bz * hw
    nk = k // _KT

    z3 = z.reshape(bz, c, hw)
    pb = proj_b.reshape(d, 1)

    ft, f2o, lseo, indo, diffo = pl.pallas_call(
        functools.partial(_qa_body, nk, nk - 1),
        grid=(bz, nk),
        in_specs=[
            pl.BlockSpec((1, c, hw), lambda b, t: (b, 0, 0)),
            pl.BlockSpec((d, c), lambda b, t: (0, 0)),
            pl.BlockSpec((d, 1), lambda b, t: (0, 0)),
            pl.BlockSpec((_KT, d), lambda b, t: (t, 0)),
        ],
        out_specs=[
            pl.BlockSpec((d, hw), lambda b, t: (0, b)),
            pl.BlockSpec((1, 1, hw), lambda b, t: (b, 0, 0)),
            pl.BlockSpec((1, 1, hw), lambda b, t: (b, 0, 0)),
            pl.BlockSpec((1, 1, hw), lambda b, t: (b, 0, 0)),
            pl.BlockSpec((1, 1), lambda b, t: (0, 0)),
        ],
        out_shape=[
            jax.ShapeDtypeStruct((d, n), jnp.float32),
            jax.ShapeDtypeStruct((bz, 1, hw), jnp.float32),
            jax.ShapeDtypeStruct((bz, 1, hw), jnp.float32),
            jax.ShapeDtypeStruct((bz, 1, hw), jnp.int32),
            jax.ShapeDtypeStruct((1, 1), jnp.float32),
        ],
        scratch_shapes=[
            pltpu.VMEM((d, hw), jnp.float32),
            pltpu.VMEM((1, hw), jnp.float32),
            pltpu.VMEM((1, hw), jnp.float32),
            pltpu.VMEM((1, hw), jnp.float32),
            pltpu.VMEM((1, hw), jnp.float32),
            pltpu.VMEM((1, hw), jnp.int32),
            pltpu.VMEM((1, 1), jnp.float32),
        ],
    )(z3, proj_w, pb, embed_w)

    zq_flat = _gather_rows(indo.reshape(n // 128, 128), embed_w)

    lp = pl.pallas_call(
        _qb_body,
        grid=(bz, nk),
        in_specs=[
            pl.BlockSpec((_KT, d), lambda b, t: (t, 0)),
            pl.BlockSpec((d, hw), lambda b, t: (0, b)),
            pl.BlockSpec((1, 1, hw), lambda b, t: (b, 0, 0)),
            pl.BlockSpec((1, 1, hw), lambda b, t: (b, 0, 0)),
        ],
        out_specs=pl.BlockSpec((1, _KT, hw), lambda b, t: (b, t, 0)),
        out_shape=jax.ShapeDtypeStruct((bz, k, hw), jnp.float32),
    )(embed_w, ft, f2o, lseo)

    zqt = pl.pallas_call(
        _qd_body,
        grid=(bz,),
        in_specs=[pl.BlockSpec((hw, d), lambda b: (b, 0))],
        out_specs=pl.BlockSpec((1, d, hw), lambda b: (b, 0, 0)),
        out_shape=jax.ShapeDtypeStruct((bz, d, hw), jnp.float32),
    )(zq_flat)

    z_q = zqt.reshape(bz, d, h, w)
    diff = diffo.reshape(())
    ind = indo.reshape(bz, h, w)
    log_priors = lp.reshape(bz, k, h, w)
    return (z_q, diff, ind, log_priors)
